# Initial kernel scaffold; baseline (speedup 1.0000x reference)
import jax
import jax.numpy as jnp
from jax.experimental import pallas as pl


def kernel(hidden_states, sim_matrix, gates, W1, W2):
    raise NotImplementedError("write your pallas kernel here")


# speedup vs baseline: 2.8658x; 2.8658x over previous
import functools

import jax
import jax.numpy as jnp
from jax import lax
from jax.experimental import pallas as pl
from jax.experimental.pallas import tpu as pltpu

T = 2048
H = 768
FF = 3072
E = 8
FFC = 768
NF = FF // FFC


def _router_body(x_ref, sim_ref, gates_ref, logits_ref, mask_ref, rw_ref):
    x = x_ref[...]
    sim = sim_ref[...]
    g = gates_ref[...]

    ssq = jnp.sum(sim * sim, axis=0, keepdims=True)
    simn = sim / jnp.maximum(jnp.sqrt(ssq), 1e-12)
    xsq = jnp.sum(x * x, axis=1, keepdims=True)
    xn = x / jnp.maximum(jnp.sqrt(xsq), 1e-12)
    logits = jax.lax.dot_general(
        xn.astype(jnp.bfloat16), simn.astype(jnp.bfloat16),
        (((1,), (0,)), ((), ())),
        preferred_element_type=jnp.float32)

    thr = jax.nn.sigmoid(g)
    gated = jnp.maximum(logits - thr, 0.0)
    maskv = (gated > 0.0).astype(jnp.float32)
    num_active = jnp.sum(maskv, axis=1, keepdims=True)

    mx = jnp.max(logits, axis=1, keepdims=True)
    ii = lax.broadcasted_iota(jnp.int32, (T, E), 1)
    cand = jnp.where(logits == mx, ii, E)
    first = jnp.min(cand, axis=1, keepdims=True)
    onehot = (ii == first).astype(jnp.float32)

    mask2 = jnp.where(num_active == 0.0, onehot, maskv)
    glm = jnp.where(mask2 > 0.0, gated, -1e9)
    m2 = jnp.max(glm, axis=1, keepdims=True)
    ex = jnp.exp(glm - m2)
    rw = ex / jnp.sum(ex, axis=1, keepdims=True)

    logits_ref[...] = logits
    mask_ref[...] = mask2
    rw_ref[...] = rw


def _expert_body(x_ref, w1_ref, w2_ref, mask_ref, rw_ref, feo_ref, final_ref):
    e = pl.program_id(0)
    f = pl.program_id(1)

    x = x_ref[...]
    w1 = w1_ref[0]
    h = jax.lax.dot_general(x, w1, (((1,), (1,)), ((), ())),
                            preferred_element_type=jnp.float32)
    a = 0.5 * h * (1.0 + lax.erf(h * 0.7071067811865476))
    w2 = w2_ref[0]
    y = jax.lax.dot_general(a.astype(jnp.bfloat16), w2,
                            (((1,), (1,)), ((), ())),
                            preferred_element_type=jnp.float32)

    @pl.when(jnp.logical_and(e == 0, f == 0))
    def _():
        final_ref[...] = jnp.zeros_like(final_ref)

    @pl.when(f == 0)
    def _():
        feo_ref[0] = y

    @pl.when(f > 0)
    def _():
        feo_ref[0] += y

    @pl.when(f == NF - 1)
    def _():
        eo = (lax.broadcasted_iota(jnp.int32, (E, 1), 0) == e
              ).astype(jnp.float32)
        mcol = jnp.dot(mask_ref[...], eo,
                       preferred_element_type=jnp.float32)
        rcol = jnp.dot(rw_ref[...], eo,
                       preferred_element_type=jnp.float32)
        acc = feo_ref[0]
        masked = acc * mcol
        feo_ref[0] = masked
        final_ref[...] += rcol * masked


@jax.jit
def kernel(hidden_states, sim_matrix, gates, W1, W2):
    x = hidden_states
    gates2d = gates.reshape(1, E)

    logits, mask, rw = pl.pallas_call(
        _router_body,
        out_shape=[
            jax.ShapeDtypeStruct((T, E), jnp.float32),
            jax.ShapeDtypeStruct((T, E), jnp.float32),
            jax.ShapeDtypeStruct((T, E), jnp.float32),
        ],
    )(x, sim_matrix, gates2d)

    xb = x.astype(jnp.bfloat16)
    W1b = W1.astype(jnp.bfloat16)
    W2b = W2.astype(jnp.bfloat16)
    feo_eth, final = pl.pallas_call(
        _expert_body,
        grid=(E, NF),
        in_specs=[
            pl.BlockSpec((T, H), lambda e, f: (0, 0)),
            pl.BlockSpec((1, FFC, H), lambda e, f: (e, f, 0)),
            pl.BlockSpec((1, H, FFC), lambda e, f: (e, 0, f)),
            pl.BlockSpec((T, E), lambda e, f: (0, 0)),
            pl.BlockSpec((T, E), lambda e, f: (0, 0)),
        ],
        out_specs=[
            pl.BlockSpec((1, T, H), lambda e, f: (e, 0, 0)),
            pl.BlockSpec((T, H), lambda e, f: (0, 0)),
        ],
        out_shape=[
            jax.ShapeDtypeStruct((E, T, H), jnp.float32),
            jax.ShapeDtypeStruct((T, H), jnp.float32),
        ],
        compiler_params=pltpu.CompilerParams(
            dimension_semantics=("arbitrary", "arbitrary"),
            vmem_limit_bytes=100 * 1024 * 1024,
        ),
    )(xb, W1b, W2b, mask, rw)

    feo = jnp.transpose(feo_eth, (1, 0, 2))
    return (final, feo, logits, mask)
